# Initial kernel scaffold; baseline (speedup 1.0000x reference)
import jax
import jax.numpy as jnp
from jax.experimental import pallas as pl


def kernel(x, params):
    raise NotImplementedError("write your pallas kernel here")


# speedup vs baseline: 1.6576x; 1.6576x over previous
import jax
import jax.numpy as jnp
from jax.experimental import pallas as pl
from jax.experimental.pallas import tpu as pltpu

_B = 128
_D = 512
_NHEAD = 8
_DH = _D // _NHEAD
_FF = 256
_OUT = (32, 32)


def _mm(a, b):
    return jax.lax.dot_general(a, b, (((1,), (0,)), ((), ())),
                               preferred_element_type=jnp.float32)


def _mm_t(a, b):
    return jax.lax.dot_general(a, b, (((1,), (1,)), ((), ())),
                               preferred_element_type=jnp.float32)


def _layernorm(x, g, b, eps=1e-5):
    m = jnp.mean(x, axis=-1, keepdims=True)
    v = jnp.mean((x - m) ** 2, axis=-1, keepdims=True)
    return (x - m) / jnp.sqrt(v + eps) * g + b


def _softmax(s):
    m = jnp.max(s, axis=-1, keepdims=True)
    e = jnp.exp(s - m)
    return e / jnp.sum(e, axis=-1, keepdims=True)


def _mha_heads(q, k, v):
    outs = []
    for h in range(_NHEAD):
        sl = slice(h * _DH, (h + 1) * _DH)
        s = _mm_t(q[:, sl], k[:, sl])
        w = _softmax(s)
        outs.append(_mm(w, v[:, sl]))
    return jnp.concatenate(outs, axis=-1)


def _vitt_kernel(x_ref,
                 sa_wq, sa_wk, sa_wv, sa_wo, sa_bq, sa_bk, sa_bv, sa_bo,
                 ca_wq, ca_wk, ca_wv, ca_wo, ca_bq, ca_bk, ca_bv, ca_bo,
                 w1, b1, w2, b2,
                 g1, be1, g2, be2, g3, be3,
                 wr1, br1, wr2, br2,
                 out_ref,
                 r_ref, x1_ref, qc_ref, outs_ref):
    x = x_ref[:]
    scale = jnp.float32(_DH ** -0.5)

    qs = (_mm(x, sa_wq[:]) + sa_bq[:]) * scale
    ks = _mm(x, sa_wk[:]) + sa_bk[:]
    vs = _mm(x, sa_wv[:]) + sa_bv[:]
    sa_out = _mm(_mha_heads(qs, ks, vs), sa_wo[:]) + sa_bo[:]
    x1 = _layernorm(x + sa_out, g1[:], be1[:])
    x1_ref[:] = x1
    qc_ref[:] = (_mm(x1, ca_wq[:]) + ca_bq[:]) * scale
    r_ref[:] = jnp.zeros((_B, _D), jnp.float32)

    def step(t, carry):
        r = r_ref[:]
        x1v = x1_ref[:]
        k = _mm(r, ca_wk[:]) + ca_bk[:]
        v = _mm(r, ca_wv[:]) + ca_bv[:]
        att = _mha_heads(qc_ref[:], k, v)
        ca_out = _mm(att, ca_wo[:]) + ca_bo[:]
        x2 = _layernorm(x1v + ca_out, g2[:], be2[:])
        ff = _mm(jnp.maximum(_mm(x2, w1[:]) + b1[:], 0.0), w2[:]) + b2[:]
        out = _layernorm(x2 + ff, g3[:], be3[:])
        r_new = r + out
        r_ref[:] = r_new
        outs_ref[pl.ds(t, 1), :, :] = r_new[0:1, :].reshape(1, 1, _D)
        return carry

    jax.lax.fori_loop(0, _B, step, 0)

    outs = outs_ref[:].reshape(_B, _D)
    h = _mm(outs, wr1[:]) + br1[:]
    out_ref[:] = _mm(h, wr2[:]) + br2[:]


def kernel(x, params):
    sa, ca = params["sa"], params["ca"]
    row = lambda a: a.reshape(1, -1)
    args = (
        x,
        sa["Wq"], sa["Wk"], sa["Wv"], sa["Wo"],
        row(sa["bq"]), row(sa["bk"]), row(sa["bv"]), row(sa["bo"]),
        ca["Wq"], ca["Wk"], ca["Wv"], ca["Wo"],
        row(ca["bq"]), row(ca["bk"]), row(ca["bv"]), row(ca["bo"]),
        params["W1"], row(params["b1"]), params["W2"], row(params["b2"]),
        row(params["g1"]), row(params["be1"]),
        row(params["g2"]), row(params["be2"]),
        row(params["g3"]), row(params["be3"]),
        params["Wr1"], row(params["br1"]), params["Wr2"], row(params["br2"]),
    )
    out = pl.pallas_call(
        _vitt_kernel,
        out_shape=jax.ShapeDtypeStruct((_B, _OUT[0] * _OUT[1]), jnp.float32),
        scratch_shapes=[
            pltpu.VMEM((_B, _D), jnp.float32),
            pltpu.VMEM((_B, _D), jnp.float32),
            pltpu.VMEM((_B, _D), jnp.float32),
            pltpu.VMEM((_B, 1, _D), jnp.float32),
        ],
    )(*args)
    return out.reshape(_B, *_OUT)
